# baseline (speedup 1.0000x reference)
import jax
import jax.numpy as jnp
from jax.experimental import pallas as pl

_B = 1024
_S = 4096
_I = 256
_BJ = 512
_NJ = _S // _BJ


def _esn_tile(inputs_ref, states_ref, win_ref, wres_ref, out_ref):
    t = pl.program_id(0)
    z = jnp.dot(states_ref[...], wres_ref[...],
                preferred_element_type=jnp.float32)
    z = z + jnp.dot(inputs_ref[...], win_ref[...],
                    preferred_element_type=jnp.float32)
    cand = jnp.tanh(z)
    sj = states_ref[:, pl.ds(t * _BJ, _BJ)]
    out_ref[...] = sj + (cand - sj)


def kernel(inputs, states, Win, Wres):
    return pl.pallas_call(
        _esn_tile,
        grid=(_NJ,),
        in_specs=[
            pl.BlockSpec((_B, _I), lambda t: (0, 0)),
            pl.BlockSpec((_B, _S), lambda t: (0, 0)),
            pl.BlockSpec((_I, _BJ), lambda t: (0, t)),
            pl.BlockSpec((_S, _BJ), lambda t: (0, t)),
        ],
        out_specs=pl.BlockSpec((_B, _BJ), lambda t: (0, t)),
        out_shape=jax.ShapeDtypeStruct((_B, _S), jnp.float32),
    )(inputs, states, Win, Wres)
